# baseline (speedup 1.0000x reference)
import functools

import jax
import jax.numpy as jnp
from jax import lax
from jax.experimental import pallas as pl
from jax.experimental.pallas import tpu as pltpu
from jax.experimental.pallas import tpu_sc as plsc

N = 49152
E = 155648
HD = 128
NF = 128
EF = 16
PER = 76

RB = 512
EB = 2432

NC = 2
NS = 16
L = 16


def _enc_body(x_ref, deg_ref, wne_ref, bne_ref, wc0_ref, wres_ref, bres_ref,
              g1_ref, r0_ref):
    h0 = jnp.maximum(jnp.dot(x_ref[...], wne_ref[...],
                             preferred_element_type=jnp.float32) + bne_ref[...], 0.0)
    dinv = lax.rsqrt(deg_ref[...] + 1.0)
    g1_ref[...] = jnp.dot(h0, wc0_ref[...],
                          preferred_element_type=jnp.float32) * dinv
    r0_ref[...] = jnp.dot(h0, wres_ref[...],
                          preferred_element_type=jnp.float32) + bres_ref[...]


def _tc_encoder(x, deg, W_ne, b_ne, W_c0, W_res, b_res):
    grid = (N // RB,)
    blk_row = pl.BlockSpec((RB, HD), lambda i: (i, 0))
    blk_deg = pl.BlockSpec((RB, 1), lambda i: (i, 0))
    blk_w = pl.BlockSpec((HD, HD), lambda i: (0, 0))
    blk_b = pl.BlockSpec((1, HD), lambda i: (0, 0))
    return pl.pallas_call(
        _enc_body,
        grid=grid,
        in_specs=[pl.BlockSpec((RB, NF), lambda i: (i, 0)), blk_deg, blk_w,
                  blk_b, blk_w, blk_w, blk_b],
        out_specs=[blk_row, blk_row],
        out_shape=[jax.ShapeDtypeStruct((N, HD), jnp.float32),
                   jax.ShapeDtypeStruct((N, HD), jnp.float32)],
    )(x, deg, W_ne, b_ne, W_c0, W_res, b_res)


def _upd_body(r_ref, agg_ref, deg_ref, bc_ref, wnext_ref,
              wres_ref, bres_ref, gnext_ref, rnext_ref):
    dinv = lax.rsqrt(deg_ref[...] + 1.0)
    conv = dinv * agg_ref[...] + bc_ref[...]
    h_new = jnp.maximum(conv + r_ref[...], 0.0)
    nxt = jnp.dot(h_new, wnext_ref[...], preferred_element_type=jnp.float32)
    gnext_ref[...] = nxt * dinv
    rnext_ref[...] = jnp.dot(h_new, wres_ref[...],
                             preferred_element_type=jnp.float32) + bres_ref[...]


def _tc_update(r, agg, deg, b_c, W_next, W_res, b_res):
    grid = (N // RB,)
    blk_row = pl.BlockSpec((RB, HD), lambda i: (i, 0))
    blk_deg = pl.BlockSpec((RB, 1), lambda i: (i, 0))
    blk_w = pl.BlockSpec((HD, HD), lambda i: (0, 0))
    blk_b = pl.BlockSpec((1, HD), lambda i: (0, 0))
    return pl.pallas_call(
        _upd_body,
        grid=grid,
        in_specs=[blk_row, blk_row, blk_deg, blk_b, blk_w, blk_w, blk_b],
        out_specs=[blk_row, blk_row],
        out_shape=[jax.ShapeDtypeStruct((N, HD), jnp.float32),
                   jax.ShapeDtypeStruct((N, HD), jnp.float32)],
    )(r, agg, deg, b_c, W_next, W_res, b_res)


def _fin_body(r_ref, agg_ref, deg_ref, bc_ref, wp_ref, bp_ref,
              p1_ref, p2_ref):
    dinv = lax.rsqrt(deg_ref[...] + 1.0)
    conv = dinv * agg_ref[...] + bc_ref[...]
    h_new = jnp.maximum(conv + r_ref[...], 0.0)
    q = jnp.dot(h_new, wp_ref[...], preferred_element_type=jnp.float32)
    q = q + bp_ref[...]
    p1_ref[...] = jnp.reshape(q[:, 0:1], (1, RB // 128, 128))
    p2_ref[...] = jnp.reshape(q[:, 1:2], (1, RB // 128, 128))


def _tc_final(r, agg, deg, b_c, wp, bp):
    grid = (N // RB,)
    blk_row = pl.BlockSpec((RB, HD), lambda i: (i, 0))
    blk_deg = pl.BlockSpec((RB, 1), lambda i: (i, 0))
    blk_b = pl.BlockSpec((1, HD), lambda i: (0, 0))
    blk_out = pl.BlockSpec((1, RB // 128, 128), lambda i: (i, 0, 0))
    return pl.pallas_call(
        _fin_body,
        grid=grid,
        in_specs=[blk_row, blk_row, blk_deg, blk_b,
                  pl.BlockSpec((HD, 2), lambda i: (0, 0)),
                  pl.BlockSpec((1, 2), lambda i: (0, 0))],
        out_specs=[blk_out, blk_out],
        out_shape=[jax.ShapeDtypeStruct((N // RB, RB // 128, 128), jnp.float32),
                   jax.ShapeDtypeStruct((N // RB, RB // 128, 128), jnp.float32)],
    )(r, agg, deg, b_c, wp, bp)


def _edge_body(ea_ref, base_ref, w1_ref, w3_ref, p3_ref):
    t = jnp.maximum(jnp.dot(ea_ref[...], w1_ref[...],
                            preferred_element_type=jnp.float32) + base_ref[...], 0.0)
    p3c = jnp.dot(t, w3_ref[...], preferred_element_type=jnp.float32)
    p3_ref[...] = jnp.reshape(p3c, (1, EB // 128, 128))


def _tc_edge(edge_attr, base16, W1, w3):
    grid = (E // EB,)
    return pl.pallas_call(
        _edge_body,
        grid=grid,
        in_specs=[pl.BlockSpec((EB, EF), lambda i: (i, 0)),
                  pl.BlockSpec((EB, HD), lambda i: (0, 0)),
                  pl.BlockSpec((EF, HD), lambda i: (0, 0)),
                  pl.BlockSpec((HD, 1), lambda i: (0, 0))],
        out_specs=pl.BlockSpec((1, EB // 128, 128), lambda i: (i, 0, 0)),
        out_shape=jax.ShapeDtypeStruct((E // EB, EB // 128, 128), jnp.float32),
    )(edge_attr, base16, W1, w3)


def _sc_mesh():
    return plsc.VectorSubcoreMesh(core_axis_name="c", subcore_axis_name="s")


def _sc_degree(col, ones128):
    ept = E // NC // NS
    nrows = ept // 128
    npt = N // NS

    @functools.partial(
        pl.kernel,
        out_type=jax.ShapeDtypeStruct((NC, N), jnp.float32),
        mesh=_sc_mesh(),
        scratch_types=[
            pltpu.VMEM((ept,), jnp.int32),
            pltpu.VMEM((128,), jnp.int32),
            pltpu.VMEM((128,), jnp.float32),
            pltpu.VMEM((npt,), jnp.float32),
            pltpu.VMEM_SHARED((N,), jnp.float32),
        ],
    )
    def k(col_hbm, ones_hbm, deg_hbm, col_t, cst, ones_v, zb, deg_sh):
        cid = lax.axis_index("c")
        sid = lax.axis_index("s")
        pltpu.sync_copy(ones_hbm, ones_v)
        z = jnp.zeros((L,), jnp.float32)

        def zfill(i, _):
            zb[pl.ds(i * L, L)] = z
            return 0
        lax.fori_loop(0, npt // L, zfill, 0)
        pltpu.sync_copy(zb, deg_sh.at[pl.ds(sid * npt, npt)])
        plsc.subcore_barrier()
        base = (cid * NS + sid) * ept
        pltpu.sync_copy(col_hbm.at[pl.ds(base, ept)], col_t)

        def chunk(j, _):
            for t in range(128 // L):
                cst[pl.ds(t * L, L)] = col_t[pl.ds(j * 128 + t * L, L)]
            pltpu.sync_copy(ones_v, deg_sh.at[cst], add=True)
            return 0
        lax.fori_loop(0, nrows, chunk, 0)
        plsc.subcore_barrier()
        pltpu.sync_copy(deg_sh.at[pl.ds(sid * npt, npt)],
                        deg_hbm.at[cid].at[pl.ds(sid * npt, npt)])

    return k(col, ones128)


NPASS = 2
NR = N // (NC * NPASS)
AGG_ROWS = NR + 512
ZR = 32
EGRP = 2432
CH = 64


def _sc_aggregate(g, col, row):
    ept = E // NS
    sl = NR // NS

    @functools.partial(
        pl.kernel,
        out_type=jax.ShapeDtypeStruct((N, HD), jnp.float32),
        mesh=_sc_mesh(),
        scratch_types=[
            pltpu.VMEM((EGRP,), jnp.int32),
            pltpu.VMEM((EGRP,), jnp.int32),
            pltpu.VMEM((CH,), jnp.int32),
            pltpu.VMEM((CH,), jnp.int32),
            pltpu.VMEM((CH,), jnp.int32),
            pltpu.VMEM((CH,), jnp.int32),
            pltpu.VMEM((CH, HD), jnp.float32),
            pltpu.VMEM((CH, HD), jnp.float32),
            pltpu.VMEM_SHARED((AGG_ROWS, HD), jnp.float32),
            pltpu.SemaphoreType.DMA,
            pltpu.SemaphoreType.DMA,
        ],
    )
    def k(g_hbm, col_hbm, row_hbm, agg_hbm,
          col_c, row_c, rst0, rst1, cst0, cst1, rows_b0, rows_b1,
          agg_sh, sem0, sem1):
        cid = lax.axis_index("c")
        sid = lax.axis_index("s")
        dump = NR + sid * 16 + lax.iota(jnp.int32, L)
        for p in range(NPASS):
            start = (cid * NPASS + p) * NR
            ibase = pl.multiple_of(sid * sl, 32)
            pltpu.sync_copy(
                g_hbm.at[pl.ds(pl.multiple_of(start + sid * sl, 32), sl)],
                agg_sh.at[pl.ds(ibase, sl)])
            plsc.subcore_barrier()
            for big in range(ept // EGRP):
                ebase = pl.multiple_of(sid * ept + big * EGRP, 8)
                pltpu.sync_copy(col_hbm.at[pl.ds(ebase, EGRP)], col_c)
                pltpu.sync_copy(row_hbm.at[pl.ds(ebase, EGRP)], row_c)
                rsts = (rst0, rst1)
                csts = (cst0, cst1)
                bufs = (rows_b0, rows_b1)
                sems = (sem0, sem1)
                nch = EGRP // CH

                def stage(c, b):
                    for t in range(CH // L):
                        cv = col_c[pl.ds(c * CH + t * L, L)]
                        lc = cv - start
                        m = (lc >= 0) & (lc < NR)
                        csts[b][pl.ds(t * L, L)] = jnp.where(m, lc, dump)
                        rsts[b][pl.ds(t * L, L)] = row_c[pl.ds(c * CH + t * L, L)]

                for b in range(2):
                    stage(b, b)
                    pltpu.async_copy(g_hbm.at[rsts[b]], bufs[b], sems[b])

                def pipe(jj, _):
                    for b in range(2):
                        c = jj * 2 + b
                        pltpu.make_async_copy(g_hbm.at[rsts[b]], bufs[b],
                                              sems[b]).wait()
                        pltpu.sync_copy(bufs[b], agg_sh.at[csts[b]], add=True)
                        stage(c + 2, b)
                        pltpu.async_copy(g_hbm.at[rsts[b]], bufs[b], sems[b])
                    return 0
                lax.fori_loop(0, nch // 2 - 1, pipe, 0)
                for b in range(2):
                    pltpu.make_async_copy(g_hbm.at[rsts[b]], bufs[b],
                                          sems[b]).wait()
                    pltpu.sync_copy(bufs[b], agg_sh.at[csts[b]], add=True)
            plsc.subcore_barrier()
            pltpu.sync_copy(
                agg_sh.at[pl.ds(pl.multiple_of(sid * sl, 32), sl)],
                agg_hbm.at[pl.ds(pl.multiple_of(start + sid * sl, 32), sl)])
            plsc.subcore_barrier()

    return k(g, col, row)


def _sc_decode(p1, p2, p3, row, col):
    ept = E // (NC * NS)

    @functools.partial(
        pl.kernel,
        out_type=jax.ShapeDtypeStruct((E,), jnp.float32),
        mesh=_sc_mesh(),
        scratch_types=[
            pltpu.VMEM((ept,), jnp.int32),
            pltpu.VMEM((ept,), jnp.int32),
            pltpu.VMEM((ept,), jnp.float32),
            pltpu.VMEM((ept,), jnp.float32),
            pltpu.VMEM((128,), jnp.float32),
            pltpu.VMEM((128,), jnp.float32),
            pltpu.VMEM((128,), jnp.float32),
            pltpu.VMEM((128,), jnp.float32),
            pltpu.SemaphoreType.DMA,
            pltpu.SemaphoreType.DMA,
            pltpu.SemaphoreType.DMA,
            pltpu.SemaphoreType.DMA,
        ],
    )
    def k(p1_hbm, p2_hbm, p3_hbm, row_hbm, col_hbm, out_hbm,
          row_t, col_t, p3_t, out_t, a0, a1, b0, b1, sa0, sa1, sb0, sb1):
        cid = lax.axis_index("c")
        sid = lax.axis_index("s")
        wid = sid * NC + cid
        base = wid * ept
        pltpu.sync_copy(row_hbm.at[pl.ds(base, ept)], row_t)
        pltpu.sync_copy(col_hbm.at[pl.ds(base, ept)], col_t)
        pltpu.sync_copy(p3_hbm.at[pl.ds(base, ept)], p3_t)
        abufs = (a0, a1)
        bbufs = (b0, b1)
        sas = (sa0, sa1)
        sbs = (sb0, sb1)
        nch = ept // 128

        def start(c, b):
            pltpu.async_copy(p1_hbm.at[row_t.at[pl.ds(c * 128, 128)]],
                             abufs[b], sas[b])
            pltpu.async_copy(p2_hbm.at[col_t.at[pl.ds(c * 128, 128)]],
                             bbufs[b], sbs[b])

        def finish(c, b):
            pltpu.make_async_copy(p1_hbm.at[row_t.at[pl.ds(c * 128, 128)]],
                                  abufs[b], sas[b]).wait()
            pltpu.make_async_copy(p2_hbm.at[col_t.at[pl.ds(c * 128, 128)]],
                                  bbufs[b], sbs[b]).wait()
            for t in range(128 // L):
                s = pl.ds(t * L, L)
                j_s = pl.ds(c * 128 + t * L, L)
                out_t[j_s] = abufs[b][s] + bbufs[b][s] + p3_t[j_s]

        for b in range(2):
            start(b, b)

        def chunk(jj, _):
            for b in range(2):
                c = jj * 2 + b
                finish(c, b)
                start(c + 2, b)
            return 0
        lax.fori_loop(0, nch // 2 - 1, chunk, 0)
        for b in range(2):
            finish(nch - 2 + b, b)
        pltpu.sync_copy(out_t, out_hbm.at[pl.ds(base, ept)])

    return k(p1, p2, p3, row, col)


def kernel(x, edge_attr, edge_index, batch, W_ne, b_ne, W_ee, b_ee, W_res,
           b_res, W_c0, b_c0, W_c1, b_c1, W_ep, b_ep):
    f32 = jnp.float32
    row = edge_index[0]
    col = edge_index[1]
    nb_residual = (batch.max() + 1 - (E // PER)).astype(f32)

    W1 = W_ee[:EF]
    W2 = W_ee[EF:]
    base = W2 + b_ee[None, :] + nb_residual * jnp.sum(W2, axis=0, keepdims=True)
    base16 = jnp.tile(base, (EB // PER, 1))
    wp = jnp.stack([W_ep[0:HD, 0], W_ep[HD:2 * HD, 0]], axis=1)
    bp = jnp.array([[1.0, 0.0]], f32) * b_ep[0]
    w3 = W_ep[2 * HD:3 * HD]

    b_ne2 = b_ne.reshape(1, HD)
    b_res2 = b_res.reshape(1, HD)
    b_c02 = b_c0.reshape(1, HD)
    b_c12 = b_c1.reshape(1, HD)

    ones128 = jnp.ones((128,), f32)

    deg2 = _sc_degree(col, ones128)
    deg = (deg2[0] + deg2[1]).reshape(N, 1)

    g1, r0 = _tc_encoder(x, deg, W_ne, b_ne2, W_c0, W_res, b_res2)

    p3 = _tc_edge(edge_attr, base16, W1, w3).reshape(E)

    agg1 = _sc_aggregate(g1, col, row)
    g2, r1 = _tc_update(r0, agg1, deg, b_c02, W_c1, W_res, b_res2)

    agg2 = _sc_aggregate(g2, col, row)
    p1c, p2c = _tc_final(r1, agg2, deg, b_c12, wp, bp)

    return _sc_decode(p1c.reshape(N), p2c.reshape(N), p3, row, col)


# speedup vs baseline: 1.0180x; 1.0180x over previous
import functools

import jax
import jax.numpy as jnp
from jax import lax
from jax.experimental import pallas as pl
from jax.experimental.pallas import tpu as pltpu
from jax.experimental.pallas import tpu_sc as plsc

N = 49152
E = 155648
HD = 128
NF = 128
EF = 16
PER = 76

RB = 512
EB = 2432

NC = 2
NS = 16
L = 16


def _enc_body(x_ref, deg_ref, wne_ref, bne_ref, wc0_ref, wres_ref, bres_ref,
              g1_ref, r0_ref):
    h0 = jnp.maximum(jnp.dot(x_ref[...], wne_ref[...],
                             preferred_element_type=jnp.float32) + bne_ref[...], 0.0)
    dinv = lax.rsqrt(deg_ref[...] + 1.0)
    g1_ref[...] = jnp.dot(h0, wc0_ref[...],
                          preferred_element_type=jnp.float32) * dinv
    r0_ref[...] = jnp.dot(h0, wres_ref[...],
                          preferred_element_type=jnp.float32) + bres_ref[...]


def _tc_encoder(x, deg, W_ne, b_ne, W_c0, W_res, b_res):
    grid = (N // RB,)
    blk_row = pl.BlockSpec((RB, HD), lambda i: (i, 0))
    blk_deg = pl.BlockSpec((RB, 1), lambda i: (i, 0))
    blk_w = pl.BlockSpec((HD, HD), lambda i: (0, 0))
    blk_b = pl.BlockSpec((1, HD), lambda i: (0, 0))
    return pl.pallas_call(
        _enc_body,
        grid=grid,
        in_specs=[pl.BlockSpec((RB, NF), lambda i: (i, 0)), blk_deg, blk_w,
                  blk_b, blk_w, blk_w, blk_b],
        out_specs=[blk_row, blk_row],
        out_shape=[jax.ShapeDtypeStruct((N, HD), jnp.float32),
                   jax.ShapeDtypeStruct((N, HD), jnp.float32)],
    )(x, deg, W_ne, b_ne, W_c0, W_res, b_res)


def _upd_body(r_ref, agg_ref, deg_ref, bc_ref, wnext_ref,
              wres_ref, bres_ref, gnext_ref, rnext_ref):
    dinv = lax.rsqrt(deg_ref[...] + 1.0)
    conv = dinv * agg_ref[...] + bc_ref[...]
    h_new = jnp.maximum(conv + r_ref[...], 0.0)
    nxt = jnp.dot(h_new, wnext_ref[...], preferred_element_type=jnp.float32)
    gnext_ref[...] = nxt * dinv
    rnext_ref[...] = jnp.dot(h_new, wres_ref[...],
                             preferred_element_type=jnp.float32) + bres_ref[...]


def _tc_update(r, agg, deg, b_c, W_next, W_res, b_res):
    grid = (N // RB,)
    blk_row = pl.BlockSpec((RB, HD), lambda i: (i, 0))
    blk_deg = pl.BlockSpec((RB, 1), lambda i: (i, 0))
    blk_w = pl.BlockSpec((HD, HD), lambda i: (0, 0))
    blk_b = pl.BlockSpec((1, HD), lambda i: (0, 0))
    return pl.pallas_call(
        _upd_body,
        grid=grid,
        in_specs=[blk_row, blk_row, blk_deg, blk_b, blk_w, blk_w, blk_b],
        out_specs=[blk_row, blk_row],
        out_shape=[jax.ShapeDtypeStruct((N, HD), jnp.float32),
                   jax.ShapeDtypeStruct((N, HD), jnp.float32)],
    )(r, agg, deg, b_c, W_next, W_res, b_res)


def _fin_body(r_ref, agg_ref, deg_ref, bc_ref, wp_ref, bp_ref,
              p1_ref, p2_ref):
    dinv = lax.rsqrt(deg_ref[...] + 1.0)
    conv = dinv * agg_ref[...] + bc_ref[...]
    h_new = jnp.maximum(conv + r_ref[...], 0.0)
    q = jnp.dot(h_new, wp_ref[...], preferred_element_type=jnp.float32)
    q = q + bp_ref[...]
    p1_ref[...] = jnp.reshape(q[:, 0:1], (1, RB // 128, 128))
    p2_ref[...] = jnp.reshape(q[:, 1:2], (1, RB // 128, 128))


def _tc_final(r, agg, deg, b_c, wp, bp):
    grid = (N // RB,)
    blk_row = pl.BlockSpec((RB, HD), lambda i: (i, 0))
    blk_deg = pl.BlockSpec((RB, 1), lambda i: (i, 0))
    blk_b = pl.BlockSpec((1, HD), lambda i: (0, 0))
    blk_out = pl.BlockSpec((1, RB // 128, 128), lambda i: (i, 0, 0))
    return pl.pallas_call(
        _fin_body,
        grid=grid,
        in_specs=[blk_row, blk_row, blk_deg, blk_b,
                  pl.BlockSpec((HD, 2), lambda i: (0, 0)),
                  pl.BlockSpec((1, 2), lambda i: (0, 0))],
        out_specs=[blk_out, blk_out],
        out_shape=[jax.ShapeDtypeStruct((N // RB, RB // 128, 128), jnp.float32),
                   jax.ShapeDtypeStruct((N // RB, RB // 128, 128), jnp.float32)],
    )(r, agg, deg, b_c, wp, bp)


def _edge_body(ea_ref, base_ref, w1_ref, w3_ref, p3_ref):
    t = jnp.maximum(jnp.dot(ea_ref[...], w1_ref[...],
                            preferred_element_type=jnp.float32) + base_ref[...], 0.0)
    p3c = jnp.dot(t, w3_ref[...], preferred_element_type=jnp.float32)
    p3_ref[...] = jnp.reshape(p3c, (1, EB // 128, 128))


def _tc_edge(edge_attr, base16, W1, w3):
    grid = (E // EB,)
    return pl.pallas_call(
        _edge_body,
        grid=grid,
        in_specs=[pl.BlockSpec((EB, EF), lambda i: (i, 0)),
                  pl.BlockSpec((EB, HD), lambda i: (0, 0)),
                  pl.BlockSpec((EF, HD), lambda i: (0, 0)),
                  pl.BlockSpec((HD, 1), lambda i: (0, 0))],
        out_specs=pl.BlockSpec((1, EB // 128, 128), lambda i: (i, 0, 0)),
        out_shape=jax.ShapeDtypeStruct((E // EB, EB // 128, 128), jnp.float32),
    )(edge_attr, base16, W1, w3)


def _sc_mesh():
    return plsc.VectorSubcoreMesh(core_axis_name="c", subcore_axis_name="s")


def _sc_degree(col, ones128):
    ept = E // NC // NS
    nrows = ept // 128
    npt = N // NS

    @functools.partial(
        pl.kernel,
        out_type=jax.ShapeDtypeStruct((NC, N), jnp.float32),
        mesh=_sc_mesh(),
        scratch_types=[
            pltpu.VMEM((ept,), jnp.int32),
            pltpu.VMEM((128,), jnp.int32),
            pltpu.VMEM((128,), jnp.float32),
            pltpu.VMEM((npt,), jnp.float32),
            pltpu.VMEM_SHARED((N,), jnp.float32),
        ],
    )
    def k(col_hbm, ones_hbm, deg_hbm, col_t, cst, ones_v, zb, deg_sh):
        cid = lax.axis_index("c")
        sid = lax.axis_index("s")
        pltpu.sync_copy(ones_hbm, ones_v)
        z = jnp.zeros((L,), jnp.float32)

        def zfill(i, _):
            zb[pl.ds(i * L, L)] = z
            return 0
        lax.fori_loop(0, npt // L, zfill, 0)
        pltpu.sync_copy(zb, deg_sh.at[pl.ds(sid * npt, npt)])
        plsc.subcore_barrier()
        base = (cid * NS + sid) * ept
        pltpu.sync_copy(col_hbm.at[pl.ds(base, ept)], col_t)

        def chunk(j, _):
            for t in range(128 // L):
                cst[pl.ds(t * L, L)] = col_t[pl.ds(j * 128 + t * L, L)]
            pltpu.sync_copy(ones_v, deg_sh.at[cst], add=True)
            return 0
        lax.fori_loop(0, nrows, chunk, 0)
        plsc.subcore_barrier()
        pltpu.sync_copy(deg_sh.at[pl.ds(sid * npt, npt)],
                        deg_hbm.at[cid].at[pl.ds(sid * npt, npt)])

    return k(col, ones128)


NPASS = 2
NR = N // (NC * NPASS)
AGG_ROWS = NR + 512
ZR = 32
EGRP = 4864
CH = 64


def _sc_aggregate(g, col, row):
    ept = E // NS
    sl = NR // NS

    @functools.partial(
        pl.kernel,
        out_type=jax.ShapeDtypeStruct((N, HD), jnp.float32),
        mesh=_sc_mesh(),
        scratch_types=[
            pltpu.VMEM((EGRP,), jnp.int32),
            pltpu.VMEM((EGRP,), jnp.int32),
            pltpu.VMEM((CH,), jnp.int32),
            pltpu.VMEM((CH,), jnp.int32),
            pltpu.VMEM((CH,), jnp.int32),
            pltpu.VMEM((CH,), jnp.int32),
            pltpu.VMEM((CH, HD), jnp.float32),
            pltpu.VMEM((CH, HD), jnp.float32),
            pltpu.VMEM_SHARED((AGG_ROWS, HD), jnp.float32),
            pltpu.SemaphoreType.DMA,
            pltpu.SemaphoreType.DMA,
        ],
    )
    def k(g_hbm, col_hbm, row_hbm, agg_hbm,
          col_c, row_c, rst0, rst1, cst0, cst1, rows_b0, rows_b1,
          agg_sh, sem0, sem1):
        cid = lax.axis_index("c")
        sid = lax.axis_index("s")
        dump = NR + sid * 16 + lax.iota(jnp.int32, L)
        for p in range(NPASS):
            start = (cid * NPASS + p) * NR
            ibase = pl.multiple_of(sid * sl, 32)
            pltpu.sync_copy(
                g_hbm.at[pl.ds(pl.multiple_of(start + sid * sl, 32), sl)],
                agg_sh.at[pl.ds(ibase, sl)])
            plsc.subcore_barrier()
            for big in range(ept // EGRP):
                ebase = pl.multiple_of(sid * ept + big * EGRP, 8)
                pltpu.sync_copy(col_hbm.at[pl.ds(ebase, EGRP)], col_c)
                pltpu.sync_copy(row_hbm.at[pl.ds(ebase, EGRP)], row_c)
                rsts = (rst0, rst1)
                csts = (cst0, cst1)
                bufs = (rows_b0, rows_b1)
                sems = (sem0, sem1)
                nch = EGRP // CH

                def stage(c, b):
                    for t in range(CH // L):
                        cv = col_c[pl.ds(c * CH + t * L, L)]
                        lc = cv - start
                        m = (lc >= 0) & (lc < NR)
                        csts[b][pl.ds(t * L, L)] = jnp.where(m, lc, dump)
                        rsts[b][pl.ds(t * L, L)] = row_c[pl.ds(c * CH + t * L, L)]

                for b in range(2):
                    stage(b, b)
                    pltpu.async_copy(g_hbm.at[rsts[b]], bufs[b], sems[b])

                def pipe(jj, _):
                    for b in range(2):
                        c = jj * 2 + b
                        pltpu.make_async_copy(g_hbm.at[rsts[b]], bufs[b],
                                              sems[b]).wait()
                        pltpu.sync_copy(bufs[b], agg_sh.at[csts[b]], add=True)
                        stage(c + 2, b)
                        pltpu.async_copy(g_hbm.at[rsts[b]], bufs[b], sems[b])
                    return 0
                lax.fori_loop(0, nch // 2 - 1, pipe, 0)
                for b in range(2):
                    pltpu.make_async_copy(g_hbm.at[rsts[b]], bufs[b],
                                          sems[b]).wait()
                    pltpu.sync_copy(bufs[b], agg_sh.at[csts[b]], add=True)
            plsc.subcore_barrier()
            pltpu.sync_copy(
                agg_sh.at[pl.ds(pl.multiple_of(sid * sl, 32), sl)],
                agg_hbm.at[pl.ds(pl.multiple_of(start + sid * sl, 32), sl)])
            plsc.subcore_barrier()

    return k(g, col, row)


def _sc_decode(p1, p2, p3, row, col):
    ept = E // (NC * NS)

    @functools.partial(
        pl.kernel,
        out_type=jax.ShapeDtypeStruct((E,), jnp.float32),
        mesh=_sc_mesh(),
        scratch_types=[
            pltpu.VMEM((ept,), jnp.int32),
            pltpu.VMEM((ept,), jnp.int32),
            pltpu.VMEM((ept,), jnp.float32),
            pltpu.VMEM((ept,), jnp.float32),
            pltpu.VMEM((128,), jnp.float32),
            pltpu.VMEM((128,), jnp.float32),
            pltpu.VMEM((128,), jnp.float32),
            pltpu.VMEM((128,), jnp.float32),
            pltpu.SemaphoreType.DMA,
            pltpu.SemaphoreType.DMA,
            pltpu.SemaphoreType.DMA,
            pltpu.SemaphoreType.DMA,
        ],
    )
    def k(p1_hbm, p2_hbm, p3_hbm, row_hbm, col_hbm, out_hbm,
          row_t, col_t, p3_t, out_t, a0, a1, b0, b1, sa0, sa1, sb0, sb1):
        cid = lax.axis_index("c")
        sid = lax.axis_index("s")
        wid = sid * NC + cid
        base = wid * ept
        pltpu.sync_copy(row_hbm.at[pl.ds(base, ept)], row_t)
        pltpu.sync_copy(col_hbm.at[pl.ds(base, ept)], col_t)
        pltpu.sync_copy(p3_hbm.at[pl.ds(base, ept)], p3_t)
        abufs = (a0, a1)
        bbufs = (b0, b1)
        sas = (sa0, sa1)
        sbs = (sb0, sb1)
        nch = ept // 128

        def start(c, b):
            pltpu.async_copy(p1_hbm.at[row_t.at[pl.ds(c * 128, 128)]],
                             abufs[b], sas[b])
            pltpu.async_copy(p2_hbm.at[col_t.at[pl.ds(c * 128, 128)]],
                             bbufs[b], sbs[b])

        def finish(c, b):
            pltpu.make_async_copy(p1_hbm.at[row_t.at[pl.ds(c * 128, 128)]],
                                  abufs[b], sas[b]).wait()
            pltpu.make_async_copy(p2_hbm.at[col_t.at[pl.ds(c * 128, 128)]],
                                  bbufs[b], sbs[b]).wait()
            for t in range(128 // L):
                s = pl.ds(t * L, L)
                j_s = pl.ds(c * 128 + t * L, L)
                out_t[j_s] = abufs[b][s] + bbufs[b][s] + p3_t[j_s]

        for b in range(2):
            start(b, b)

        def chunk(jj, _):
            for b in range(2):
                c = jj * 2 + b
                finish(c, b)
                start(c + 2, b)
            return 0
        lax.fori_loop(0, nch // 2 - 1, chunk, 0)
        for b in range(2):
            finish(nch - 2 + b, b)
        pltpu.sync_copy(out_t, out_hbm.at[pl.ds(base, ept)])

    return k(p1, p2, p3, row, col)


def kernel(x, edge_attr, edge_index, batch, W_ne, b_ne, W_ee, b_ee, W_res,
           b_res, W_c0, b_c0, W_c1, b_c1, W_ep, b_ep):
    f32 = jnp.float32
    row = edge_index[0]
    col = edge_index[1]
    nb_residual = (batch.max() + 1 - (E // PER)).astype(f32)

    W1 = W_ee[:EF]
    W2 = W_ee[EF:]
    base = W2 + b_ee[None, :] + nb_residual * jnp.sum(W2, axis=0, keepdims=True)
    base16 = jnp.tile(base, (EB // PER, 1))
    wp = jnp.stack([W_ep[0:HD, 0], W_ep[HD:2 * HD, 0]], axis=1)
    bp = jnp.array([[1.0, 0.0]], f32) * b_ep[0]
    w3 = W_ep[2 * HD:3 * HD]

    b_ne2 = b_ne.reshape(1, HD)
    b_res2 = b_res.reshape(1, HD)
    b_c02 = b_c0.reshape(1, HD)
    b_c12 = b_c1.reshape(1, HD)

    ones128 = jnp.ones((128,), f32)

    deg2 = _sc_degree(col, ones128)
    deg = (deg2[0] + deg2[1]).reshape(N, 1)

    g1, r0 = _tc_encoder(x, deg, W_ne, b_ne2, W_c0, W_res, b_res2)

    p3 = _tc_edge(edge_attr, base16, W1, w3).reshape(E)

    agg1 = _sc_aggregate(g1, col, row)
    g2, r1 = _tc_update(r0, agg1, deg, b_c02, W_c1, W_res, b_res2)

    agg2 = _sc_aggregate(g2, col, row)
    p1c, p2c = _tc_final(r1, agg2, deg, b_c12, wp, bp)

    return _sc_decode(p1c.reshape(N), p2c.reshape(N), p3, row, col)
